# baseline (speedup 1.0000x reference)
import functools

import jax
import jax.numpy as jnp
from jax import lax
from jax.experimental import pallas as pl
from jax.experimental.pallas import tpu as pltpu
from jax.experimental.pallas import tpu_sc as plsc

D = 64
B = 4096
S = 50
BT = 128
SG = 5
NSTEP = S // SG

_info = plsc.get_sparse_core_info()
NC, NS = _info.num_cores, _info.num_subcores
NW = NC * NS

_mesh = plsc.VectorSubcoreMesh(core_axis_name="c", subcore_axis_name="s")


def _transpose_tile(rows_v, j, t_v):
    iota = lax.iota(jnp.int32, 16)
    dtv = [((g * 16) + iota) >> 3 for g in range(4)]
    drv = [((g * 16) + iota) & 7 for g in range(4)]

    def body(bi, carry):
        row0 = j * BT + bi * 8
        for u in range(8):
            blv = jnp.zeros((16,), jnp.int32) + (bi * 8 + u)
            vals = [rows_v[row0 + u, pl.ds(g * 16, 16)] for g in range(4)]
            for g in range(4):
                plsc.store_scatter(t_v, [dtv[g], drv[g], blv], vals[g])
        return carry

    lax.fori_loop(0, BT // 8, body, 0)


@functools.partial(
    pl.kernel,
    mesh=_mesh,
    out_type=jax.ShapeDtypeStruct((S, 8, NW, 8, BT), jnp.float32),
    scratch_types=[
        pltpu.VMEM((SG * BT,), jnp.int32),
        pltpu.VMEM((SG * BT,), jnp.int32),
        pltpu.VMEM((SG * BT, D), jnp.float32),
        pltpu.VMEM((SG * BT, D), jnp.float32),
        pltpu.VMEM((8, 8, BT + 1), jnp.float32),
        pltpu.VMEM((8, 8, BT + 1), jnp.float32),
        pltpu.VMEM((8, 8, BT + 1), jnp.float32),
        pltpu.VMEM((8, 8, BT + 1), jnp.float32),
        pltpu.VMEM((8, 8, BT + 1), jnp.float32),
        pltpu.SemaphoreType.DMA,
        pltpu.SemaphoreType.DMA,
        pltpu.SemaphoreType.DMA,
        pltpu.SemaphoreType.DMA,
        pltpu.SemaphoreType.DMA,
        pltpu.SemaphoreType.DMA,
        pltpu.SemaphoreType.DMA,
    ],
    compiler_params=pltpu.CompilerParams(
        use_tc_tiling_on_sc=False, needs_layout_passes=False),
)
def _gather_kernel(table_hbm, ids_hbm, out_hbm, idx0, idx1,
                   rows0, rows1, t0, t1, t2, t3, t4,
                   sg0, sg1, sw0, sw1, sw2, sw3, sw4):
    w = lax.axis_index("s") * NC + lax.axis_index("c")

    ts = (t0, t1, t2, t3, t4)
    sws = (sw0, sw1, sw2, sw3, sw4)

    def gather_start(idx_v, rows_v, sem):
        return pltpu.async_copy(table_hbm.at[idx_v], rows_v, sem)

    def g_wait(idx_v, rows_v, sem):
        pltpu.make_async_copy(table_hbm.at[idx_v], rows_v, sem).wait()

    def wb_start(tpar, s, sem):
        return pltpu.async_copy(
            ts[tpar].at[:, :, pl.ds(0, BT)], out_hbm.at[s, :, w], sem)

    def wb_wait(tpar, s, sem):
        pltpu.make_async_copy(
            ts[tpar].at[:, :, pl.ds(0, BT)], out_hbm.at[s, :, w], sem).wait()

    def do_step(m, p_is_odd, s0, idx_v, rows_v, sg_cur):
        g_wait(idx_v, rows_v, sg_cur)
        for j in range(SG):
            s = s0 + j
            if not p_is_odd:
                @pl.when(m > 0)
                def _():
                    wb_wait(j, s - SG, sws[j])
            else:
                wb_wait(j, s - SG, sws[j])
            _transpose_tile(rows_v, j, ts[j])
            wb_start(j, s, sws[j])

    def load_idx(s0, idx_v):
        for c in range(SG):
            pltpu.sync_copy(ids_hbm.at[s0 + c, pl.ds(w * BT, BT)],
                            idx_v.at[pl.ds(c * BT, BT)])

    load_idx(0, idx0)
    gather_start(idx0, rows0, sg0)

    def body(m, carry):
        s0 = 2 * m * SG
        load_idx(s0 + SG, idx1)
        gather_start(idx1, rows1, sg1)
        do_step(m, False, s0, idx0, rows0, sg0)

        @pl.when(m < NSTEP // 2 - 1)
        def _():
            load_idx(s0 + 2 * SG, idx0)
            gather_start(idx0, rows0, sg0)

        do_step(m, True, s0 + SG, idx1, rows1, sg1)
        return carry

    lax.fori_loop(0, NSTEP // 2, body, 0)
    for j in range(SG):
        wb_wait(j, S - SG + j, sws[j])


def kernel(instrument_ids, table):
    ids_t = instrument_ids.astype(jnp.int32).T
    out5 = _gather_kernel(table, ids_t)
    return out5.transpose(2, 4, 0, 1, 3).reshape(B, S, D)


# speedup vs baseline: 1.1342x; 1.1342x over previous
import functools

import jax
import jax.numpy as jnp
from jax import lax
from jax.experimental import pallas as pl
from jax.experimental.pallas import tpu as pltpu
from jax.experimental.pallas import tpu_sc as plsc

D = 64
B = 4096
S = 50
BT = 128
SG = 5
NSTEP = S // SG

_info = plsc.get_sparse_core_info()
NC, NS = _info.num_cores, _info.num_subcores
NW = NC * NS

_mesh = plsc.VectorSubcoreMesh(core_axis_name="c", subcore_axis_name="s")


def _transpose_tile(rows_v, j, t_v):
    iota = lax.iota(jnp.int32, 16)
    dtv = [((g * 16) + iota) >> 3 for g in range(4)]
    drv = [((g * 16) + iota) & 7 for g in range(4)]

    def body(bi, carry):
        row0 = j * BT + bi * 8
        for u in range(8):
            blv = jnp.zeros((16,), jnp.int32) + (bi * 8 + u)
            vals = [rows_v[row0 + u, pl.ds(g * 16, 16)] for g in range(4)]
            for g in range(4):
                plsc.store_scatter(t_v, [dtv[g], drv[g], blv], vals[g])
        return carry

    lax.fori_loop(0, BT // 8, body, 0)


@functools.partial(
    pl.kernel,
    mesh=_mesh,
    out_type=jax.ShapeDtypeStruct((S, 8, NW, 8, BT), jnp.float32),
    scratch_types=[
        pltpu.VMEM((SG * BT,), jnp.int32),
        pltpu.VMEM((SG * BT,), jnp.int32),
        pltpu.VMEM((SG * BT, D), jnp.float32),
        pltpu.VMEM((SG * BT, D), jnp.float32),
        pltpu.VMEM((8, 8, BT + 1), jnp.float32),
        pltpu.VMEM((8, 8, BT + 1), jnp.float32),
        pltpu.VMEM((8, 8, BT + 1), jnp.float32),
        pltpu.VMEM((8, 8, BT + 1), jnp.float32),
        pltpu.VMEM((8, 8, BT + 1), jnp.float32),
        pltpu.SemaphoreType.DMA,
        pltpu.SemaphoreType.DMA,
        pltpu.SemaphoreType.DMA,
        pltpu.SemaphoreType.DMA,
        pltpu.SemaphoreType.DMA,
        pltpu.SemaphoreType.DMA,
        pltpu.SemaphoreType.DMA,
        pltpu.SemaphoreType.DMA,
        pltpu.SemaphoreType.DMA,
    ],
    compiler_params=pltpu.CompilerParams(
        use_tc_tiling_on_sc=False, needs_layout_passes=False),
)
def _gather_kernel(table_hbm, ids_hbm, out_hbm, idx0, idx1,
                   rows0, rows1, t0, t1, t2, t3, t4,
                   sg0, sg1, si0, si1, sw0, sw1, sw2, sw3, sw4):
    w = lax.axis_index("s") * NC + lax.axis_index("c")

    ts = (t0, t1, t2, t3, t4)
    sws = (sw0, sw1, sw2, sw3, sw4)

    def gather_start(idx_v, rows_v, sem):
        return pltpu.async_copy(table_hbm.at[idx_v], rows_v, sem)

    def g_wait(idx_v, rows_v, sem):
        pltpu.make_async_copy(table_hbm.at[idx_v], rows_v, sem).wait()

    def wb_start(tpar, s, sem):
        return pltpu.async_copy(
            ts[tpar].at[:, :, pl.ds(0, BT)], out_hbm.at[s, :, w], sem)

    def wb_wait(tpar, s, sem):
        pltpu.make_async_copy(
            ts[tpar].at[:, :, pl.ds(0, BT)], out_hbm.at[s, :, w], sem).wait()

    def do_step(m, p_is_odd, s0, idx_v, rows_v, sg_cur, prefetch):
        g_wait(idx_v, rows_v, sg_cur)
        if prefetch is not None:
            pred, p_next, sem = prefetch

            @pl.when(pred)
            def _():
                idx_start(p_next, idx_v, sem)
        for j in range(SG):
            s = s0 + j
            if not p_is_odd:
                @pl.when(m > 0)
                def _():
                    wb_wait(j, s - SG, sws[j])
            else:
                wb_wait(j, s - SG, sws[j])
            _transpose_tile(rows_v, j, ts[j])
            wb_start(j, s, sws[j])

    def idx_start(p, idx_v, sem):
        return pltpu.async_copy(ids_hbm.at[p, w], idx_v, sem)

    def idx_wait(p, idx_v, sem):
        pltpu.make_async_copy(ids_hbm.at[p, w], idx_v, sem).wait()

    idx_start(0, idx0, si0)
    idx_wait(0, idx0, si0)
    gather_start(idx0, rows0, sg0)
    idx_start(1, idx1, si1)

    def body(m, carry):
        p = 2 * m
        s0 = p * SG
        last = NSTEP // 2 - 1
        idx_wait(p + 1, idx1, si1)
        gather_start(idx1, rows1, sg1)
        do_step(m, False, s0, idx0, rows0, sg0, (m < last, p + 2, si0))

        @pl.when(m < last)
        def _():
            idx_wait(p + 2, idx0, si0)
            gather_start(idx0, rows0, sg0)

        do_step(m, True, s0 + SG, idx1, rows1, sg1, (m < last, p + 3, si1))
        return carry

    lax.fori_loop(0, NSTEP // 2, body, 0)
    for j in range(SG):
        wb_wait(j, S - SG + j, sws[j])


def kernel(instrument_ids, table):
    ids3 = (instrument_ids.astype(jnp.int32).T
            .reshape(NSTEP, SG, NW, BT)
            .transpose(0, 2, 1, 3)
            .reshape(NSTEP, NW, SG * BT))
    out5 = _gather_kernel(table, ids3)
    return out5.transpose(2, 4, 0, 1, 3).reshape(B, S, D)
